# initial kernel scaffold (unmeasured)
import jax
import jax.numpy as jnp
from jax import lax
from jax.experimental import pallas as pl
from jax.experimental.pallas import tpu as pltpu

N_DEV = 4


def kernel(x, w_mat):
    m_per, k = x.shape
    _, n_per = w_mat.shape
    half = m_per // 2

    def _mm_relu(a, w):
        y = jnp.dot(a, w, preferred_element_type=jnp.float32)
        return jnp.maximum(y, 0.0)

    def body(x_ref, w_ref, out_ref, cw_buf, ccw_buf,
             cw_send, cw_recv, ccw_send, ccw_recv):
        my = lax.axis_index("i")
        right = (my + 1) % N_DEV
        left = (my + N_DEV - 1) % N_DEV

        barrier = pltpu.get_barrier_semaphore()
        for nbr in (left, right):
            pl.semaphore_signal(
                barrier, inc=1,
                device_id=(nbr,), device_id_type=pl.DeviceIdType.MESH,
            )
        pl.semaphore_wait(barrier, 2)

        cw_buf[0, :, :] = x_ref[:half, :]
        ccw_buf[0, :, :] = x_ref[half:, :]

        out_ref[pl.ds(my * m_per, m_per), :] = _mm_relu(x_ref[...], w_ref[...])

        for h in range(N_DEV - 1):
            s, r = h % 2, (h + 1) % 2
            cw = pltpu.make_async_remote_copy(
                src_ref=cw_buf.at[s],
                dst_ref=cw_buf.at[r],
                send_sem=cw_send.at[h],
                recv_sem=cw_recv.at[h],
                device_id=(right,),
                device_id_type=pl.DeviceIdType.MESH,
            )
            ccw = pltpu.make_async_remote_copy(
                src_ref=ccw_buf.at[s],
                dst_ref=ccw_buf.at[r],
                send_sem=ccw_send.at[h],
                recv_sem=ccw_recv.at[h],
                device_id=(left,),
                device_id_type=pl.DeviceIdType.MESH,
            )
            cw.start()
            ccw.start()
            cw.wait()
            ccw.wait()

            org_cw = (my + N_DEV - 1 - h) % N_DEV
            org_ccw = (my + h + 1) % N_DEV
            out_ref[pl.ds(org_cw * m_per, half), :] = _mm_relu(
                cw_buf[r], w_ref[...])
            out_ref[pl.ds(org_ccw * m_per + half, half), :] = _mm_relu(
                ccw_buf[r], w_ref[...])

    return pl.pallas_call(
        body,
        out_shape=jax.ShapeDtypeStruct((N_DEV * m_per, n_per), jnp.float32),
        in_specs=[
            pl.BlockSpec(memory_space=pltpu.VMEM),
            pl.BlockSpec(memory_space=pltpu.VMEM),
        ],
        out_specs=pl.BlockSpec(memory_space=pltpu.VMEM),
        scratch_shapes=[
            pltpu.VMEM((2, half, k), jnp.float32),
            pltpu.VMEM((2, half, k), jnp.float32),
            pltpu.SemaphoreType.DMA((N_DEV - 1,)),
            pltpu.SemaphoreType.DMA((N_DEV - 1,)),
            pltpu.SemaphoreType.DMA((N_DEV - 1,)),
            pltpu.SemaphoreType.DMA((N_DEV - 1,)),
        ],
        compiler_params=pltpu.CompilerParams(collective_id=0),
    )(x, w_mat)


# baseline (device time: 301936 ns/iter reference)
import jax
import jax.numpy as jnp
from jax import lax
from jax.experimental import pallas as pl
from jax.experimental.pallas import tpu as pltpu

N_DEV = 4


def kernel(x, w_mat):
    m_per, k = x.shape
    _, n_per = w_mat.shape
    half = m_per // 2

    def _mm_relu(a, w):
        y = jnp.dot(a, w, preferred_element_type=jnp.float32)
        return jnp.maximum(y, 0.0)

    def body(x_ref, w_ref, out_ref, cw_buf, ccw_buf,
             local_sems, cw_send, cw_recv, ccw_send, ccw_recv):
        my = lax.axis_index("i")
        right = (my + 1) % N_DEV
        left = (my + N_DEV - 1) % N_DEV

        barrier = pltpu.get_barrier_semaphore()
        for nbr in (left, right):
            pl.semaphore_signal(
                barrier, inc=1,
                device_id=(nbr,), device_id_type=pl.DeviceIdType.MESH,
            )
        pl.semaphore_wait(barrier, 2)

        stage_cw = pltpu.make_async_copy(
            x_ref.at[pl.ds(0, half), :], cw_buf.at[0], local_sems.at[0])
        stage_ccw = pltpu.make_async_copy(
            x_ref.at[pl.ds(half, half), :], ccw_buf.at[0], local_sems.at[1])
        stage_cw.start()
        stage_ccw.start()
        stage_cw.wait()
        stage_ccw.wait()

        def hop_rdmas(h):
            s, r = h % 2, (h + 1) % 2
            cw = pltpu.make_async_remote_copy(
                src_ref=cw_buf.at[s],
                dst_ref=cw_buf.at[r],
                send_sem=cw_send.at[h],
                recv_sem=cw_recv.at[h],
                device_id=(right,),
                device_id_type=pl.DeviceIdType.MESH,
            )
            ccw = pltpu.make_async_remote_copy(
                src_ref=ccw_buf.at[s],
                dst_ref=ccw_buf.at[r],
                send_sem=ccw_send.at[h],
                recv_sem=ccw_recv.at[h],
                device_id=(left,),
                device_id_type=pl.DeviceIdType.MESH,
            )
            return cw, ccw

        cw0, ccw0 = hop_rdmas(0)
        cw0.start()
        ccw0.start()
        out_ref[pl.ds(my * m_per, half), :] = _mm_relu(cw_buf[0], w_ref[...])
        out_ref[pl.ds(my * m_per + half, half), :] = _mm_relu(
            ccw_buf[0], w_ref[...])

        hop = {0: (cw0, ccw0)}
        for h in range(N_DEV - 1):
            r = (h + 1) % 2
            cw_h, ccw_h = hop[h]
            cw_h.wait()
            ccw_h.wait()
            if h + 1 < N_DEV - 1:
                cw_n, ccw_n = hop_rdmas(h + 1)
                cw_n.start()
                ccw_n.start()
                hop[h + 1] = (cw_n, ccw_n)

            org_cw = (my + N_DEV - 1 - h) % N_DEV
            org_ccw = (my + h + 1) % N_DEV
            out_ref[pl.ds(org_cw * m_per, half), :] = _mm_relu(
                cw_buf[r], w_ref[...])
            out_ref[pl.ds(org_ccw * m_per + half, half), :] = _mm_relu(
                ccw_buf[r], w_ref[...])

    return pl.pallas_call(
        body,
        out_shape=jax.ShapeDtypeStruct((N_DEV * m_per, n_per), jnp.float32),
        in_specs=[
            pl.BlockSpec(memory_space=pl.ANY),
            pl.BlockSpec(memory_space=pltpu.VMEM),
        ],
        out_specs=pl.BlockSpec(memory_space=pltpu.VMEM),
        scratch_shapes=[
            pltpu.VMEM((2, half, k), jnp.float32),
            pltpu.VMEM((2, half, k), jnp.float32),
            pltpu.SemaphoreType.DMA((2,)),
            pltpu.SemaphoreType.DMA((N_DEV - 1,)),
            pltpu.SemaphoreType.DMA((N_DEV - 1,)),
            pltpu.SemaphoreType.DMA((N_DEV - 1,)),
            pltpu.SemaphoreType.DMA((N_DEV - 1,)),
        ],
        compiler_params=pltpu.CompilerParams(
            collective_id=0,
            vmem_limit_bytes=100 * 1024 * 1024,
        ),
    )(x, w_mat)


# device time: 205726 ns/iter; 1.4677x vs baseline; 1.4677x over previous
import jax
import jax.numpy as jnp
from jax import lax
from jax.experimental import pallas as pl
from jax.experimental.pallas import tpu as pltpu

N_DEV = 4


def kernel(x, w_mat):
    m_per, k = x.shape
    _, n_per = w_mat.shape
    wh = n_per // 2

    def _mm_relu(a, w):
        y = jnp.dot(a, w, preferred_element_type=jnp.float32)
        return jnp.maximum(y, 0.0)

    def body(x_ref, w_ref, out_ref, cw_buf, ccw_buf, res_buf,
             cw_send, cw_recv, ccw_send, ccw_recv, res_send, res_recv):
        my = lax.axis_index("i")
        right = (my + 1) % N_DEV
        left = (my + N_DEV - 1) % N_DEV
        diag = (my + 2) % N_DEV

        barrier = pltpu.get_barrier_semaphore()
        for nbr in (left, right, diag):
            pl.semaphore_signal(
                barrier, inc=1,
                device_id=(nbr,), device_id_type=pl.DeviceIdType.MESH,
            )
        pl.semaphore_wait(barrier, 3)

        cw_buf[0, :, :] = w_ref[:, :wh]
        ccw_buf[0, :, :] = w_ref[:, wh:]

        def hop_rdmas(h):
            s, r = h % 2, (h + 1) % 2
            cw = pltpu.make_async_remote_copy(
                src_ref=cw_buf.at[s], dst_ref=cw_buf.at[r],
                send_sem=cw_send.at[h], recv_sem=cw_recv.at[h],
                device_id=(right,), device_id_type=pl.DeviceIdType.MESH,
            )
            ccw = pltpu.make_async_remote_copy(
                src_ref=ccw_buf.at[s], dst_ref=ccw_buf.at[r],
                send_sem=ccw_send.at[h], recv_sem=ccw_recv.at[h],
                device_id=(left,), device_id_type=pl.DeviceIdType.MESH,
            )
            return cw, ccw

        def result_rdma(slot, dest, col0):
            return pltpu.make_async_remote_copy(
                src_ref=res_buf.at[slot],
                dst_ref=out_ref.at[pl.ds(my * m_per, m_per),
                                   pl.ds(col0, wh)],
                send_sem=res_send.at[slot],
                recv_sem=res_recv.at[slot],
                device_id=(dest,), device_id_type=pl.DeviceIdType.MESH,
            )

        cw0, ccw0 = hop_rdmas(0)
        cw0.start()
        ccw0.start()
        out_ref[pl.ds(my * m_per, m_per), :] = _mm_relu(
            x_ref[...], w_ref[...])

        sends = []
        hop = {0: (cw0, ccw0)}
        for h in range(N_DEV - 1):
            r = (h + 1) % 2
            cw_h, ccw_h = hop[h]
            cw_h.wait()
            ccw_h.wait()
            if h + 1 < N_DEV - 1:
                nxt = hop_rdmas(h + 1)
                nxt[0].start()
                nxt[1].start()
                hop[h + 1] = nxt

            dest_cw = (my + N_DEV - 1 - h) % N_DEV
            dest_ccw = (my + h + 1) % N_DEV
            res_buf[2 * h, :, :] = _mm_relu(x_ref[...], cw_buf[r])
            s_cw = result_rdma(2 * h, dest_cw, 0)
            s_cw.start()
            res_buf[2 * h + 1, :, :] = _mm_relu(x_ref[...], ccw_buf[r])
            s_ccw = result_rdma(2 * h + 1, dest_ccw, wh)
            s_ccw.start()
            sends += [s_cw, s_ccw]

        for h in range(N_DEV - 1):
            org_cw = (my + h + 1) % N_DEV
            org_ccw = (my + N_DEV - 1 - h) % N_DEV
            for slot, org, col0 in (
                (2 * h, org_cw, 0),
                (2 * h + 1, org_ccw, wh),
            ):
                recv = pltpu.make_async_remote_copy(
                    src_ref=res_buf.at[slot],
                    dst_ref=out_ref.at[pl.ds(org * m_per, m_per),
                                       pl.ds(col0, wh)],
                    send_sem=res_send.at[slot],
                    recv_sem=res_recv.at[slot],
                    device_id=(org,),
                    device_id_type=pl.DeviceIdType.MESH,
                )
                recv.wait_recv()
        for s in sends:
            s.wait_send()

    return pl.pallas_call(
        body,
        out_shape=jax.ShapeDtypeStruct((N_DEV * m_per, n_per), jnp.float32),
        in_specs=[
            pl.BlockSpec(memory_space=pltpu.VMEM),
            pl.BlockSpec(memory_space=pltpu.VMEM),
        ],
        out_specs=pl.BlockSpec(memory_space=pltpu.VMEM),
        scratch_shapes=[
            pltpu.VMEM((2, k, wh), jnp.float32),
            pltpu.VMEM((2, k, wh), jnp.float32),
            pltpu.VMEM((2 * (N_DEV - 1), m_per, wh), jnp.float32),
            pltpu.SemaphoreType.DMA((N_DEV - 1,)),
            pltpu.SemaphoreType.DMA((N_DEV - 1,)),
            pltpu.SemaphoreType.DMA((N_DEV - 1,)),
            pltpu.SemaphoreType.DMA((N_DEV - 1,)),
            pltpu.SemaphoreType.DMA((2 * (N_DEV - 1),)),
            pltpu.SemaphoreType.DMA((2 * (N_DEV - 1),)),
        ],
        compiler_params=pltpu.CompilerParams(
            collective_id=0,
            vmem_limit_bytes=100 * 1024 * 1024,
        ),
    )(x, w_mat)


# device time: 203234 ns/iter; 1.4857x vs baseline; 1.0123x over previous
import jax
import jax.numpy as jnp
from jax import lax
from jax.experimental import pallas as pl
from jax.experimental.pallas import tpu as pltpu

N_DEV = 4
N_STREAMS = 4


def kernel(x, w_mat):
    m_per, k = x.shape
    _, n_per = w_mat.shape
    wq = n_per // N_STREAMS
    n_hops = N_DEV - 1

    def _mm_relu(a, w):
        y = jnp.dot(a, w, preferred_element_type=jnp.float32)
        return jnp.maximum(y, 0.0)

    def body(x_ref, w_ref, out_ref, w_bufs, res_buf,
             ring_send, ring_recv, res_send, res_recv):
        my = lax.axis_index("i")
        right = (my + 1) % N_DEV
        left = (my + N_DEV - 1) % N_DEV
        diag = (my + 2) % N_DEV

        barrier = pltpu.get_barrier_semaphore()
        for nbr in (left, right, diag):
            pl.semaphore_signal(
                barrier, inc=1,
                device_id=(nbr,), device_id_type=pl.DeviceIdType.MESH,
            )
        pl.semaphore_wait(barrier, 3)

        streams = [(0, 1, 0), (1, 1, wq), (2, -1, 2 * wq), (3, -1, 3 * wq)]

        def ring_rdma(st, step, col0, h):
            if h == 0:
                src = w_ref.at[:, pl.ds(col0, wq)]
            else:
                src = w_bufs.at[st, (h - 1) % 2]
            tgt = right if step == 1 else left
            return pltpu.make_async_remote_copy(
                src_ref=src,
                dst_ref=w_bufs.at[st, h % 2],
                send_sem=ring_send.at[st * n_hops + h],
                recv_sem=ring_recv.at[st * n_hops + h],
                device_id=(tgt,), device_id_type=pl.DeviceIdType.MESH,
            )

        def result_rdma(slot, dest, col0):
            return pltpu.make_async_remote_copy(
                src_ref=res_buf.at[slot],
                dst_ref=out_ref.at[pl.ds(my * m_per, m_per),
                                   pl.ds(col0, wq)],
                send_sem=res_send.at[slot],
                recv_sem=res_recv.at[slot],
                device_id=(dest,), device_id_type=pl.DeviceIdType.MESH,
            )

        hop_rdma = {}
        for st, step, col0 in streams:
            r = ring_rdma(st, step, col0, 0)
            r.start()
            hop_rdma[(st, 0)] = r
        out_ref[pl.ds(my * m_per, m_per), :] = _mm_relu(
            x_ref[...], w_ref[...])

        sends = []
        order = [streams[0], streams[2], streams[1], streams[3]]
        for h in range(n_hops):
            for st, step, col0 in order:
                hop_rdma[(st, h)].wait()
                if h + 1 < n_hops:
                    nxt = ring_rdma(st, step, col0, h + 1)
                    nxt.start()
                    hop_rdma[(st, h + 1)] = nxt
                dest = (my + N_DEV - step * (h + 1)) % N_DEV
                slot = st * n_hops + h
                res_buf[slot, :, :] = _mm_relu(
                    x_ref[...], w_bufs[st, h % 2])
                s = result_rdma(slot, dest, col0)
                s.start()
                sends.append(s)

        for st, step, col0 in streams:
            for h in range(n_hops):
                org = (my + N_DEV + step * (h + 1)) % N_DEV
                slot = st * n_hops + h
                recv = pltpu.make_async_remote_copy(
                    src_ref=res_buf.at[slot],
                    dst_ref=out_ref.at[pl.ds(org * m_per, m_per),
                                       pl.ds(col0, wq)],
                    send_sem=res_send.at[slot],
                    recv_sem=res_recv.at[slot],
                    device_id=(org,),
                    device_id_type=pl.DeviceIdType.MESH,
                )
                recv.wait_recv()
        for s in sends:
            s.wait_send()

    n_msgs = N_STREAMS * n_hops
    return pl.pallas_call(
        body,
        out_shape=jax.ShapeDtypeStruct((N_DEV * m_per, n_per), jnp.float32),
        in_specs=[
            pl.BlockSpec(memory_space=pltpu.VMEM),
            pl.BlockSpec(memory_space=pltpu.VMEM),
        ],
        out_specs=pl.BlockSpec(memory_space=pltpu.VMEM),
        scratch_shapes=[
            pltpu.VMEM((N_STREAMS, 2, k, wq), jnp.float32),
            pltpu.VMEM((n_msgs, m_per, wq), jnp.float32),
            pltpu.SemaphoreType.DMA((n_msgs,)),
            pltpu.SemaphoreType.DMA((n_msgs,)),
            pltpu.SemaphoreType.DMA((n_msgs,)),
            pltpu.SemaphoreType.DMA((n_msgs,)),
        ],
        compiler_params=pltpu.CompilerParams(
            collective_id=0,
            vmem_limit_bytes=100 * 1024 * 1024,
        ),
    )(x, w_mat)
